# baseline (speedup 1.0000x reference)
import functools

import jax
import jax.numpy as jnp
from jax import lax
from jax.experimental import pallas as pl
from jax.experimental.pallas import tpu as pltpu
from jax.experimental.pallas import tpu_sc as plsc

N = 10000
E = 320000
D = 128
R = 8
NB = 2

NC = 2
NS = 16
NW = NC * NS
EPW = E // NW
K = 80
NCHUNK = EPW // K
SBC = 25
NSB = NCHUNK // SBC
NBUF = 3
ROWS_PER_SUB = 624
TAIL_ROWS = N - ROWS_PER_SUB * NS
ZROWS = 48


def _proj_body(a_ref, feat_ref, v_ref, out_ref):
    r = pl.program_id(0)
    w = a_ref[r, 0] * v_ref[0] + a_ref[r, 1] * v_ref[1]
    out_ref[...] = jnp.dot(feat_ref[...], w, preferred_element_type=jnp.float32)


def _eidx_body(et_ref, src_ref, out_ref):
    out_ref[...] = et_ref[...] * N + src_ref[...]


def _epi_body(p_ref, f_ref, b_ref, o_ref):
    h = p_ref[0] + p_ref[1] + b_ref[...]
    o_ref[...] = jnp.maximum(h, 0.0) + f_ref[...]


def _sc_edge_body(hw_hbm, eidx_hbm, dst_hbm, norm_hbm, out_hbm,
                  eidx_v, dst_v, norm_v, rows_v, zbuf, acc_sh, sem_g, sem_s):
    cid = lax.axis_index("c")
    sid = lax.axis_index("s")
    wid = sid * NC + cid

    zvec = jnp.zeros((16,), jnp.float32)
    def zero_body(i, _):
        row = i // 8
        col = (i % 8) * 16
        zbuf[row, pl.ds(col, 16)] = zvec
        return 0
    lax.fori_loop(0, ZROWS * (D // 16), zero_body, 0)
    for t in range(ROWS_PER_SUB // ZROWS):
        base = pl.multiple_of(sid * ROWS_PER_SUB + t * ZROWS, 8)
        pltpu.sync_copy(zbuf, acc_sh.at[pl.ds(base, ZROWS)])
    @pl.when(sid == 0)
    def _zero_tail():
        pltpu.sync_copy(zbuf.at[pl.ds(0, TAIL_ROWS)],
                        acc_sh.at[pl.ds(ROWS_PER_SUB * NS, TAIL_ROWS)])
    plsc.subcore_barrier()

    def super_body(sb, _):
        pltpu.sync_copy(eidx_hbm.at[wid, sb], eidx_v)
        pltpu.sync_copy(dst_hbm.at[wid, sb], dst_v)
        pltpu.sync_copy(norm_hbm.at[wid, sb], norm_v)

        for b in range(NBUF):
            pltpu.async_copy(hw_hbm.at[eidx_v.at[b]], rows_v.at[b],
                             sem_g.at[b])

        def chunk_body(c, _):
            b = c % NBUF
            pltpu.make_async_copy(hw_hbm.at[eidx_v.at[c]], rows_v.at[b],
                                  sem_g.at[b]).wait()

            def scale_body(g, _):
                nv = norm_v[c, pl.ds(g * 16, 16)]
                for l in range(16):
                    w = nv[l]
                    e = g * 16 + l
                    for j in range(D // 16):
                        rows_v[b, e, pl.ds(j * 16, 16)] = (
                            rows_v[b, e, pl.ds(j * 16, 16)] * w)
                return 0
            lax.fori_loop(0, K // 16, scale_body, 0)

            pltpu.async_copy(rows_v.at[b], acc_sh.at[dst_v.at[c]],
                             sem_s.at[b], add=True)

            g_next = c + NBUF - 1
            bp = (c + NBUF - 1) % NBUF

            @pl.when(jnp.logical_and(c >= 1, g_next < SBC))
            def _refill():
                pltpu.make_async_copy(rows_v.at[bp],
                                      acc_sh.at[dst_v.at[c - 1]],
                                      sem_s.at[bp]).wait()
                pltpu.async_copy(hw_hbm.at[eidx_v.at[g_next]],
                                 rows_v.at[bp], sem_g.at[bp])
            return 0

        lax.fori_loop(0, SBC, chunk_body, 0)

        for j in range(NBUF):
            c = SBC - NBUF + j
            b = c % NBUF
            pltpu.make_async_copy(rows_v.at[b], acc_sh.at[dst_v.at[c]],
                                  sem_s.at[b]).wait()
        return 0

    lax.fori_loop(0, NSB, super_body, 0)
    plsc.subcore_barrier()

    rbase = pl.multiple_of(sid * ROWS_PER_SUB, 8)
    pltpu.sync_copy(acc_sh.at[pl.ds(rbase, ROWS_PER_SUB)],
                    out_hbm.at[cid, pl.ds(rbase, ROWS_PER_SUB)])
    @pl.when(sid == 0)
    def _copy_tail():
        pltpu.sync_copy(acc_sh.at[pl.ds(ROWS_PER_SUB * NS, TAIL_ROWS)],
                        out_hbm.at[cid, pl.ds(ROWS_PER_SUB * NS, TAIL_ROWS)])


def kernel(features, edge_index, edge_type, norm, V0, a0, b0, V1, a1, b1):
    del V0, a0, b0
    src = edge_index[0]
    dst = edge_index[1].reshape(NW, NSB, SBC, K)
    normf = norm.reshape(NW, NSB, SBC, K)

    BN = 1000
    hw = pl.pallas_call(
        _proj_body,
        grid=(R, N // BN),
        in_specs=[
            pl.BlockSpec(memory_space=pltpu.SMEM),
            pl.BlockSpec((BN, D), lambda r, i: (i, 0)),
            pl.BlockSpec((NB, D, D), lambda r, i: (0, 0, 0)),
        ],
        out_specs=pl.BlockSpec((BN, D), lambda r, i: (r * (N // BN) + i, 0)),
        out_shape=jax.ShapeDtypeStruct((R * N, D), jnp.float32),
    )(a1, features, V1)

    EB = 2500
    eidx = pl.pallas_call(
        _eidx_body,
        out_shape=jax.ShapeDtypeStruct((EB, E // EB), jnp.int32),
    )(edge_type.reshape(EB, E // EB), src.reshape(EB, E // EB))
    eidx = eidx.reshape(NW, NSB, SBC, K)

    mesh = plsc.VectorSubcoreMesh(core_axis_name="c", subcore_axis_name="s")
    partials = pl.kernel(
        _sc_edge_body,
        out_type=jax.ShapeDtypeStruct((NC, N, D), jnp.float32),
        mesh=mesh,
        scratch_types=[
            pltpu.VMEM((SBC, K), jnp.int32),
            pltpu.VMEM((SBC, K), jnp.int32),
            pltpu.VMEM((SBC, K), jnp.float32),
            pltpu.VMEM((NBUF, K, D), jnp.float32),
            pltpu.VMEM((ZROWS, D), jnp.float32),
            pltpu.VMEM_SHARED((N, D), jnp.float32),
            pltpu.SemaphoreType.DMA((NBUF,)),
            pltpu.SemaphoreType.DMA((NBUF,)),
        ],
    )(hw, eidx, dst, normf)

    out = pl.pallas_call(
        _epi_body,
        grid=(N // BN,),
        in_specs=[
            pl.BlockSpec((NC, BN, D), lambda i: (0, i, 0)),
            pl.BlockSpec((BN, D), lambda i: (i, 0)),
            pl.BlockSpec((1, D), lambda i: (0, 0)),
        ],
        out_specs=pl.BlockSpec((BN, D), lambda i: (i, 0)),
        out_shape=jax.ShapeDtypeStruct((N, D), jnp.float32),
    )(partials, features, b1.reshape(1, D))
    return out


# speedup vs baseline: 1.5489x; 1.5489x over previous
import functools

import jax
import jax.numpy as jnp
from jax import lax
from jax.experimental import pallas as pl
from jax.experimental.pallas import tpu as pltpu
from jax.experimental.pallas import tpu_sc as plsc

N = 10000
E = 320000
D = 128
R = 8
NB = 2

NC = 2
NS = 16
NW = NC * NS
EPW = E // NW
K = 80
NCHUNK = EPW // K
SBC = 25
NSB = NCHUNK // SBC
NBUF = 2
ROWS_PER_SUB = 624
TAIL_ROWS = N - ROWS_PER_SUB * NS
ZROWS = 48


def _proj_body(a_ref, feat_ref, v_ref, out_ref):
    r = pl.program_id(0)
    w = a_ref[r, 0] * v_ref[0] + a_ref[r, 1] * v_ref[1]
    out_ref[...] = jnp.dot(feat_ref[...], w, preferred_element_type=jnp.float32)


def _eidx_body(et_ref, src_ref, out_ref):
    out_ref[...] = et_ref[...] * N + src_ref[...]


def _epi_body(p_ref, f_ref, b_ref, o_ref):
    h = p_ref[0] + p_ref[1] + b_ref[...]
    o_ref[...] = jnp.maximum(h, 0.0) + f_ref[...]


def _sc_edge_body(hw_hbm, eidx_hbm, dst_hbm, norm_hbm, out_hbm,
                  eidx_v, dst_v, norm_v, rows_v, zbuf, acc_sh, sem_g, sem_s):
    cid = lax.axis_index("c")
    sid = lax.axis_index("s")
    wid = sid * NC + cid

    zvec = jnp.zeros((16,), jnp.float32)
    def zero_body(i, _):
        row = i // 8
        col = (i % 8) * 16
        zbuf[row, pl.ds(col, 16)] = zvec
        return 0
    lax.fori_loop(0, ZROWS * (D // 16), zero_body, 0)
    for t in range(ROWS_PER_SUB // ZROWS):
        base = pl.multiple_of(sid * ROWS_PER_SUB + t * ZROWS, 8)
        pltpu.sync_copy(zbuf, acc_sh.at[pl.ds(base, ZROWS)])
    @pl.when(sid == 0)
    def _zero_tail():
        pltpu.sync_copy(zbuf.at[pl.ds(0, TAIL_ROWS)],
                        acc_sh.at[pl.ds(ROWS_PER_SUB * NS, TAIL_ROWS)])
    plsc.subcore_barrier()

    def scale(c, cur):
        def scale_body(g, _):
            nv = norm_v[c, pl.ds(g * 16, 16)]
            for l in range(16):
                w = nv[l]
                e = g * 16 + l
                for j in range(D // 16):
                    rows_v[cur, e, pl.ds(j * 16, 16)] = (
                        rows_v[cur, e, pl.ds(j * 16, 16)] * w)
            return 0
        lax.fori_loop(0, K // 16, scale_body, 0)

    def chunk_op(c, cur, nxt, prefetch):
        pltpu.make_async_copy(hw_hbm.at[eidx_v.at[c]], rows_v.at[cur],
                              sem_g.at[cur]).wait()
        scale(c, cur)
        pltpu.async_copy(rows_v.at[cur], acc_sh.at[dst_v.at[c]],
                         sem_s.at[cur], add=True)
        if prefetch:
            @pl.when(c >= 1)
            def _drain_prev():
                pltpu.make_async_copy(rows_v.at[nxt],
                                      acc_sh.at[dst_v.at[c - 1]],
                                      sem_s.at[nxt]).wait()
            pltpu.async_copy(hw_hbm.at[eidx_v.at[c + 1]], rows_v.at[nxt],
                             sem_g.at[nxt])

    def super_body(sb, _):
        pltpu.sync_copy(eidx_hbm.at[wid, sb], eidx_v)
        pltpu.sync_copy(dst_hbm.at[wid, sb], dst_v)
        pltpu.sync_copy(norm_hbm.at[wid, sb], norm_v)

        pltpu.async_copy(hw_hbm.at[eidx_v.at[0]], rows_v.at[0], sem_g.at[0])

        def pair_body(p, _):
            chunk_op(2 * p, 0, 1, True)
            chunk_op(2 * p + 1, 1, 0, True)
            return 0
        lax.fori_loop(0, SBC // 2, pair_body, 0)

        chunk_op(SBC - 1, 0, 1, False)
        pltpu.make_async_copy(rows_v.at[1], acc_sh.at[dst_v.at[SBC - 2]],
                              sem_s.at[1]).wait()
        pltpu.make_async_copy(rows_v.at[0], acc_sh.at[dst_v.at[SBC - 1]],
                              sem_s.at[0]).wait()
        return 0

    lax.fori_loop(0, NSB, super_body, 0)
    plsc.subcore_barrier()

    rbase = pl.multiple_of(sid * ROWS_PER_SUB, 8)
    pltpu.sync_copy(acc_sh.at[pl.ds(rbase, ROWS_PER_SUB)],
                    out_hbm.at[cid, pl.ds(rbase, ROWS_PER_SUB)])
    @pl.when(sid == 0)
    def _copy_tail():
        pltpu.sync_copy(acc_sh.at[pl.ds(ROWS_PER_SUB * NS, TAIL_ROWS)],
                        out_hbm.at[cid, pl.ds(ROWS_PER_SUB * NS, TAIL_ROWS)])


def kernel(features, edge_index, edge_type, norm, V0, a0, b0, V1, a1, b1):
    del V0, a0, b0
    src = edge_index[0]
    dst = edge_index[1].reshape(NW, NSB, SBC, K)
    normf = norm.reshape(NW, NSB, SBC, K)

    BN = 1000
    hw = pl.pallas_call(
        _proj_body,
        grid=(R, N // BN),
        in_specs=[
            pl.BlockSpec(memory_space=pltpu.SMEM),
            pl.BlockSpec((BN, D), lambda r, i: (i, 0)),
            pl.BlockSpec((NB, D, D), lambda r, i: (0, 0, 0)),
        ],
        out_specs=pl.BlockSpec((BN, D), lambda r, i: (r * (N // BN) + i, 0)),
        out_shape=jax.ShapeDtypeStruct((R * N, D), jnp.float32),
    )(a1, features, V1)

    EB = 2500
    eidx = pl.pallas_call(
        _eidx_body,
        out_shape=jax.ShapeDtypeStruct((EB, E // EB), jnp.int32),
    )(edge_type.reshape(EB, E // EB), src.reshape(EB, E // EB))
    eidx = eidx.reshape(NW, NSB, SBC, K)

    mesh = plsc.VectorSubcoreMesh(core_axis_name="c", subcore_axis_name="s")
    partials = pl.kernel(
        _sc_edge_body,
        out_type=jax.ShapeDtypeStruct((NC, N, D), jnp.float32),
        mesh=mesh,
        scratch_types=[
            pltpu.VMEM((SBC, K), jnp.int32),
            pltpu.VMEM((SBC, K), jnp.int32),
            pltpu.VMEM((SBC, K), jnp.float32),
            pltpu.VMEM((NBUF, K, D), jnp.float32),
            pltpu.VMEM((ZROWS, D), jnp.float32),
            pltpu.VMEM_SHARED((N, D), jnp.float32),
            pltpu.SemaphoreType.DMA((NBUF,)),
            pltpu.SemaphoreType.DMA((NBUF,)),
        ],
    )(hw, eidx, dst, normf)

    out = pl.pallas_call(
        _epi_body,
        grid=(N // BN,),
        in_specs=[
            pl.BlockSpec((NC, BN, D), lambda i: (0, i, 0)),
            pl.BlockSpec((BN, D), lambda i: (i, 0)),
            pl.BlockSpec((1, D), lambda i: (0, 0)),
        ],
        out_specs=pl.BlockSpec((BN, D), lambda i: (i, 0)),
        out_shape=jax.ShapeDtypeStruct((N, D), jnp.float32),
    )(partials, features, b1.reshape(1, D))
    return out


# speedup vs baseline: 1.7928x; 1.1575x over previous
import functools

import jax
import jax.numpy as jnp
from jax import lax
from jax.experimental import pallas as pl
from jax.experimental.pallas import tpu as pltpu
from jax.experimental.pallas import tpu_sc as plsc

N = 10000
E = 320000
D = 128
R = 8
NB = 2

NC = 2
NS = 16
NW = NC * NS
EPW = E // NW
K = 80
NCHUNK = EPW // K
SBC = 25
NSB = NCHUNK // SBC
NBUF = 2
ROWS_PER_SUB = 624
TAIL_ROWS = N - ROWS_PER_SUB * NS
ZROWS = 48


def _proj_body(a_ref, feat_ref, v_ref, out_ref):
    r = pl.program_id(0)
    w = a_ref[r, 0] * v_ref[0] + a_ref[r, 1] * v_ref[1]
    out_ref[...] = jnp.dot(feat_ref[...], w, preferred_element_type=jnp.float32)


def _eidx_body(et_ref, src_ref, out_ref):
    out_ref[...] = et_ref[...] * N + src_ref[...]


def _epi_body(p_ref, f_ref, b_ref, o_ref):
    h = p_ref[0] + p_ref[1] + b_ref[...]
    o_ref[...] = jnp.maximum(h, 0.0) + f_ref[...]


def _sc_edge_body(hw_hbm, eidx_hbm, dst_hbm, norm_hbm, out_hbm,
                  eidx_v, dst_v, norm_v, rows_v, zbuf, acc_sh, sem_g, sem_s):
    cid = lax.axis_index("c")
    sid = lax.axis_index("s")
    wid = sid * NC + cid

    zvec = jnp.zeros((16,), jnp.float32)
    def zero_body(i, _):
        row = i // 8
        col = (i % 8) * 16
        zbuf[row, pl.ds(col, 16)] = zvec
        return 0
    lax.fori_loop(0, ZROWS * (D // 16), zero_body, 0)
    for t in range(ROWS_PER_SUB // ZROWS):
        base = pl.multiple_of(sid * ROWS_PER_SUB + t * ZROWS, 8)
        pltpu.sync_copy(zbuf, acc_sh.at[pl.ds(base, ZROWS)])
    @pl.when(sid == 0)
    def _zero_tail():
        pltpu.sync_copy(zbuf.at[pl.ds(0, TAIL_ROWS)],
                        acc_sh.at[pl.ds(ROWS_PER_SUB * NS, TAIL_ROWS)])
    plsc.subcore_barrier()

    def scale(c, cur):
        def scale_body(g, _):
            nv = norm_v[c, pl.ds(g * 16, 16)]
            for l in range(16):
                w = nv[l]
                e = g * 16 + l
                for j in range(D // 16):
                    rows_v[cur, e, pl.ds(j * 16, 16)] = (
                        rows_v[cur, e, pl.ds(j * 16, 16)] * w)
            return 0
        lax.fori_loop(0, K // 16, scale_body, 0)

    def chunk_op(c, cur, nxt, prefetch):
        pltpu.make_async_copy(hw_hbm.at[eidx_v.at[c]], rows_v.at[cur],
                              sem_g.at[cur]).wait()
        if prefetch:
            @pl.when(c >= 1)
            def _drain_prev():
                pltpu.make_async_copy(rows_v.at[nxt],
                                      acc_sh.at[dst_v.at[c - 1]],
                                      sem_s.at[nxt]).wait()
            pltpu.async_copy(hw_hbm.at[eidx_v.at[c + 1]], rows_v.at[nxt],
                             sem_g.at[nxt])
        scale(c, cur)
        pltpu.async_copy(rows_v.at[cur], acc_sh.at[dst_v.at[c]],
                         sem_s.at[cur], add=True)

    def super_body(sb, _):
        pltpu.sync_copy(eidx_hbm.at[wid, sb], eidx_v)
        pltpu.sync_copy(dst_hbm.at[wid, sb], dst_v)
        pltpu.sync_copy(norm_hbm.at[wid, sb], norm_v)

        pltpu.async_copy(hw_hbm.at[eidx_v.at[0]], rows_v.at[0], sem_g.at[0])

        def pair_body(p, _):
            chunk_op(2 * p, 0, 1, True)
            chunk_op(2 * p + 1, 1, 0, True)
            return 0
        lax.fori_loop(0, SBC // 2, pair_body, 0)

        chunk_op(SBC - 1, 0, 1, False)
        pltpu.make_async_copy(rows_v.at[1], acc_sh.at[dst_v.at[SBC - 2]],
                              sem_s.at[1]).wait()
        pltpu.make_async_copy(rows_v.at[0], acc_sh.at[dst_v.at[SBC - 1]],
                              sem_s.at[0]).wait()
        return 0

    lax.fori_loop(0, NSB, super_body, 0)
    plsc.subcore_barrier()

    rbase = pl.multiple_of(sid * ROWS_PER_SUB, 8)
    pltpu.sync_copy(acc_sh.at[pl.ds(rbase, ROWS_PER_SUB)],
                    out_hbm.at[cid, pl.ds(rbase, ROWS_PER_SUB)])
    @pl.when(sid == 0)
    def _copy_tail():
        pltpu.sync_copy(acc_sh.at[pl.ds(ROWS_PER_SUB * NS, TAIL_ROWS)],
                        out_hbm.at[cid, pl.ds(ROWS_PER_SUB * NS, TAIL_ROWS)])


def kernel(features, edge_index, edge_type, norm, V0, a0, b0, V1, a1, b1):
    del V0, a0, b0
    src = edge_index[0]
    dst = edge_index[1].reshape(NW, NSB, SBC, K)
    normf = norm.reshape(NW, NSB, SBC, K)

    BN = 1000
    hw = pl.pallas_call(
        _proj_body,
        grid=(R, N // BN),
        in_specs=[
            pl.BlockSpec(memory_space=pltpu.SMEM),
            pl.BlockSpec((BN, D), lambda r, i: (i, 0)),
            pl.BlockSpec((NB, D, D), lambda r, i: (0, 0, 0)),
        ],
        out_specs=pl.BlockSpec((BN, D), lambda r, i: (r * (N // BN) + i, 0)),
        out_shape=jax.ShapeDtypeStruct((R * N, D), jnp.float32),
    )(a1, features, V1)

    EB = 2500
    eidx = pl.pallas_call(
        _eidx_body,
        out_shape=jax.ShapeDtypeStruct((EB, E // EB), jnp.int32),
    )(edge_type.reshape(EB, E // EB), src.reshape(EB, E // EB))
    eidx = eidx.reshape(NW, NSB, SBC, K)

    mesh = plsc.VectorSubcoreMesh(core_axis_name="c", subcore_axis_name="s")
    partials = pl.kernel(
        _sc_edge_body,
        out_type=jax.ShapeDtypeStruct((NC, N, D), jnp.float32),
        mesh=mesh,
        scratch_types=[
            pltpu.VMEM((SBC, K), jnp.int32),
            pltpu.VMEM((SBC, K), jnp.int32),
            pltpu.VMEM((SBC, K), jnp.float32),
            pltpu.VMEM((NBUF, K, D), jnp.float32),
            pltpu.VMEM((ZROWS, D), jnp.float32),
            pltpu.VMEM_SHARED((N, D), jnp.float32),
            pltpu.SemaphoreType.DMA((NBUF,)),
            pltpu.SemaphoreType.DMA((NBUF,)),
        ],
    )(hw, eidx, dst, normf)

    out = pl.pallas_call(
        _epi_body,
        grid=(N // BN,),
        in_specs=[
            pl.BlockSpec((NC, BN, D), lambda i: (0, i, 0)),
            pl.BlockSpec((BN, D), lambda i: (i, 0)),
            pl.BlockSpec((1, D), lambda i: (0, 0)),
        ],
        out_specs=pl.BlockSpec((BN, D), lambda i: (i, 0)),
        out_shape=jax.ShapeDtypeStruct((N, D), jnp.float32),
    )(partials, features, b1.reshape(1, D))
    return out


# speedup vs baseline: 2.0572x; 1.1475x over previous
import jax
import jax.numpy as jnp
from jax import lax
from jax.experimental import pallas as pl
from jax.experimental.pallas import tpu as pltpu
from jax.experimental.pallas import tpu_sc as plsc

N = 10000
E = 320000
D = 128
R = 8
NB = 2

NC = 2
NS = 16
NW = NC * NS
EPW = E // NW
K = 80
NCHUNK = EPW // K
SBC = 25
SBE = SBC * K
NSB = NCHUNK // SBC
NBUF = 2
ROWS_PER_SUB = 624
TAIL_ROWS = N - ROWS_PER_SUB * NS
ZROWS = 48
BN = 2000


def _proj_body(a_ref, feat_ref, v_ref, out_ref):
    r = pl.program_id(1)
    w = a_ref[r, 0] * v_ref[0] + a_ref[r, 1] * v_ref[1]
    out_ref[...] = jnp.dot(feat_ref[...], w, preferred_element_type=jnp.float32)


def _epi_body(p_ref, f_ref, b_ref, o_ref):
    h = p_ref[0] + p_ref[1] + b_ref[...]
    o_ref[...] = jnp.maximum(h, 0.0) + f_ref[...]


def _sc_edge_body(hw_hbm, src_hbm, et_hbm, norm_hbm, dst_hbm, out_hbm,
                  et_v, src_v, norm_v, eidx_v, dst_v, rows_v, zbuf, acc_sh,
                  sem_g, sem_s):
    cid = lax.axis_index("c")
    sid = lax.axis_index("s")
    wid = sid * NC + cid

    zvec = jnp.zeros((16,), jnp.float32)
    def zero_body(i, _):
        row = i // 8
        col = (i % 8) * 16
        zbuf[row, pl.ds(col, 16)] = zvec
        return 0
    lax.fori_loop(0, ZROWS * (D // 16), zero_body, 0)
    for t in range(ROWS_PER_SUB // ZROWS):
        base = pl.multiple_of(sid * ROWS_PER_SUB + t * ZROWS, 8)
        pltpu.sync_copy(zbuf, acc_sh.at[pl.ds(base, ZROWS)])
    @pl.when(sid == 0)
    def _zero_tail():
        pltpu.sync_copy(zbuf.at[pl.ds(0, TAIL_ROWS)],
                        acc_sh.at[pl.ds(ROWS_PER_SUB * NS, TAIL_ROWS)])
    plsc.subcore_barrier()

    def scale(c, cur):
        def scale_body(g, _):
            nv = norm_v[pl.ds(c * K + g * 16, 16)]
            for l in range(16):
                w = nv[l]
                e = g * 16 + l
                for j in range(D // 16):
                    rows_v[cur, e, pl.ds(j * 16, 16)] = (
                        rows_v[cur, e, pl.ds(j * 16, 16)] * w)
            return 0
        lax.fori_loop(0, K // 16, scale_body, 0)

    def chunk_op(c, cur, nxt, prefetch):
        pltpu.make_async_copy(hw_hbm.at[eidx_v.at[pl.ds(c * K, K)]],
                              rows_v.at[cur], sem_g.at[cur]).wait()
        if prefetch:
            @pl.when(c >= 1)
            def _drain_prev():
                pltpu.make_async_copy(rows_v.at[nxt],
                                      acc_sh.at[dst_v.at[c - 1]],
                                      sem_s.at[nxt]).wait()
            pltpu.async_copy(hw_hbm.at[eidx_v.at[pl.ds((c + 1) * K, K)]],
                             rows_v.at[nxt], sem_g.at[nxt])
        scale(c, cur)
        pltpu.async_copy(rows_v.at[cur], acc_sh.at[dst_v.at[c]],
                         sem_s.at[cur], add=True)

    def super_body(sb, _):
        ebase = pl.multiple_of((wid * NSB + sb) * SBE, 8)
        pltpu.sync_copy(et_hbm.at[pl.ds(ebase, SBE)], et_v)
        pltpu.sync_copy(src_hbm.at[pl.ds(ebase, SBE)], src_v)
        pltpu.sync_copy(norm_hbm.at[pl.ds(ebase, SBE)], norm_v)
        pltpu.sync_copy(dst_hbm.at[wid, sb], dst_v)

        def eidx_body(g, _):
            o = g * 16
            eidx_v[pl.ds(o, 16)] = et_v[pl.ds(o, 16)] * N + src_v[pl.ds(o, 16)]
            return 0
        lax.fori_loop(0, SBE // 16, eidx_body, 0)

        pltpu.async_copy(hw_hbm.at[eidx_v.at[pl.ds(0, K)]], rows_v.at[0],
                         sem_g.at[0])

        def pair_body(p, _):
            chunk_op(2 * p, 0, 1, True)
            chunk_op(2 * p + 1, 1, 0, True)
            return 0
        lax.fori_loop(0, SBC // 2, pair_body, 0)

        chunk_op(SBC - 1, 0, 1, False)
        pltpu.make_async_copy(rows_v.at[1], acc_sh.at[dst_v.at[SBC - 2]],
                              sem_s.at[1]).wait()
        pltpu.make_async_copy(rows_v.at[0], acc_sh.at[dst_v.at[SBC - 1]],
                              sem_s.at[0]).wait()
        return 0

    lax.fori_loop(0, NSB, super_body, 0)
    plsc.subcore_barrier()

    rbase = pl.multiple_of(sid * ROWS_PER_SUB, 8)
    pltpu.sync_copy(acc_sh.at[pl.ds(rbase, ROWS_PER_SUB)],
                    out_hbm.at[cid, pl.ds(rbase, ROWS_PER_SUB)])
    @pl.when(sid == 0)
    def _copy_tail():
        pltpu.sync_copy(acc_sh.at[pl.ds(ROWS_PER_SUB * NS, TAIL_ROWS)],
                        out_hbm.at[cid, pl.ds(ROWS_PER_SUB * NS, TAIL_ROWS)])


def kernel(features, edge_index, edge_type, norm, V0, a0, b0, V1, a1, b1):
    del V0, a0, b0
    srcf = edge_index[0]
    dst = edge_index[1].reshape(NW, NSB, SBC, K)
    normf = norm.reshape(E)

    hw = pl.pallas_call(
        _proj_body,
        grid=(N // BN, R),
        in_specs=[
            pl.BlockSpec(memory_space=pltpu.SMEM),
            pl.BlockSpec((BN, D), lambda i, r: (i, 0)),
            pl.BlockSpec((NB, D, D), lambda i, r: (0, 0, 0)),
        ],
        out_specs=pl.BlockSpec((BN, D), lambda i, r: (r * (N // BN) + i, 0)),
        out_shape=jax.ShapeDtypeStruct((R * N, D), jnp.float32),
    )(a1, features, V1)

    mesh = plsc.VectorSubcoreMesh(core_axis_name="c", subcore_axis_name="s")
    partials = pl.kernel(
        _sc_edge_body,
        out_type=jax.ShapeDtypeStruct((NC, N, D), jnp.float32),
        mesh=mesh,
        scratch_types=[
            pltpu.VMEM((SBE,), jnp.int32),
            pltpu.VMEM((SBE,), jnp.int32),
            pltpu.VMEM((SBE,), jnp.float32),
            pltpu.VMEM((SBE,), jnp.int32),
            pltpu.VMEM((SBC, K), jnp.int32),
            pltpu.VMEM((NBUF, K, D), jnp.float32),
            pltpu.VMEM((ZROWS, D), jnp.float32),
            pltpu.VMEM_SHARED((N, D), jnp.float32),
            pltpu.SemaphoreType.DMA((NBUF,)),
            pltpu.SemaphoreType.DMA((NBUF,)),
        ],
    )(hw, srcf, edge_type, normf, dst)

    out = pl.pallas_call(
        _epi_body,
        grid=(N // BN,),
        in_specs=[
            pl.BlockSpec((NC, BN, D), lambda i: (0, i, 0)),
            pl.BlockSpec((BN, D), lambda i: (i, 0)),
            pl.BlockSpec((1, D), lambda i: (0, 0)),
        ],
        out_specs=pl.BlockSpec((BN, D), lambda i: (i, 0)),
        out_shape=jax.ShapeDtypeStruct((N, D), jnp.float32),
    )(partials, features, b1.reshape(1, D))
    return out
